# baseline (speedup 1.0000x reference)
import functools

import jax
import jax.numpy as jnp
from jax import lax
from jax.experimental import pallas as pl
from jax.experimental.pallas import tpu as pltpu
from jax.experimental.pallas import tpu_sc as plsc

B = 16384
E = 32
H1 = 256
H2 = 64


def _gather_sc(user_ids, movie_ids, user_table, movie_table):
    info = plsc.get_sparse_core_info()
    nc, ns = info.num_cores, info.num_subcores
    nw = nc * ns
    b_per_w = B // nw
    mesh = plsc.VectorSubcoreMesh(core_axis_name="c", subcore_axis_name="s")

    @functools.partial(
        pl.kernel,
        mesh=mesh,
        out_type=(
            jax.ShapeDtypeStruct((B, E), jnp.float32),
            jax.ShapeDtypeStruct((B, E), jnp.float32),
        ),
        scratch_types=[
            pltpu.VMEM((b_per_w,), jnp.int32),
            pltpu.VMEM((b_per_w, E), jnp.float32),
            pltpu.VMEM((b_per_w,), jnp.int32),
            pltpu.VMEM((b_per_w, E), jnp.float32),
            pltpu.SemaphoreType.DMA,
            pltpu.SemaphoreType.DMA,
        ],
        compiler_params=pltpu.CompilerParams(use_tc_tiling_on_sc=False),
    )
    def gather_kernel(uids_hbm, mids_hbm, utab_hbm, mtab_hbm, uout_hbm,
                      mout_hbm, uidx_v, urows_v, midx_v, mrows_v, usem, msem):
        wid = lax.axis_index("s") * nc + lax.axis_index("c")
        base = wid * b_per_w
        pltpu.sync_copy(uids_hbm.at[pl.ds(base, b_per_w)], uidx_v)
        pltpu.sync_copy(mids_hbm.at[pl.ds(base, b_per_w)], midx_v)
        cu = pltpu.async_copy(utab_hbm.at[uidx_v], urows_v, usem)
        cm = pltpu.async_copy(mtab_hbm.at[midx_v], mrows_v, msem)
        cu.wait()
        cm.wait()
        pltpu.sync_copy(urows_v, uout_hbm.at[pl.ds(base, b_per_w)])
        pltpu.sync_copy(mrows_v, mout_hbm.at[pl.ds(base, b_per_w)])

    return gather_kernel(user_ids, movie_ids, user_table, movie_table)


def _mlp_body(u_ref, m_ref, w1u_ref, w1m_ref, b1_ref, w2_ref, b2_ref,
              w3_ref, b3_ref, out_ref):
    h = jnp.dot(u_ref[...], w1u_ref[...], preferred_element_type=jnp.float32)
    h += jnp.dot(m_ref[...], w1m_ref[...], preferred_element_type=jnp.float32)
    h = jnp.maximum(h + b1_ref[...], 0.0)
    h = jnp.dot(h, w2_ref[...], preferred_element_type=jnp.float32)
    h = jnp.maximum(h + b2_ref[...], 0.0)
    out_ref[...] = jnp.sum(h * w3_ref[...], axis=1, keepdims=True) + b3_ref[...]


def _mlp_tc(u_emb, m_emb, W1, b1, W2, b2, W3, b3):
    blk = 2048
    w1u = W1[:E]
    w1m = W1[E:]
    b1r = b1.reshape(1, H1)
    b2r = b2.reshape(1, H2)
    w3r = W3.reshape(1, H2)
    b3r = b3.reshape(1, 1)
    return pl.pallas_call(
        _mlp_body,
        grid=(B // blk,),
        in_specs=[
            pl.BlockSpec((blk, E), lambda i: (i, 0)),
            pl.BlockSpec((blk, E), lambda i: (i, 0)),
            pl.BlockSpec((E, H1), lambda i: (0, 0)),
            pl.BlockSpec((E, H1), lambda i: (0, 0)),
            pl.BlockSpec((1, H1), lambda i: (0, 0)),
            pl.BlockSpec((H1, H2), lambda i: (0, 0)),
            pl.BlockSpec((1, H2), lambda i: (0, 0)),
            pl.BlockSpec((1, H2), lambda i: (0, 0)),
            pl.BlockSpec((1, 1), lambda i: (0, 0)),
        ],
        out_specs=pl.BlockSpec((blk, 1), lambda i: (i, 0)),
        out_shape=jax.ShapeDtypeStruct((B, 1), jnp.float32),
    )(u_emb, m_emb, w1u, w1m, b1r, W2, b2r, w3r, b3r)


def kernel(user_ids, movie_ids, user_table, movie_table, W1, b1, W2, b2, W3, b3):
    uids = user_ids.astype(jnp.int32)
    mids = movie_ids.astype(jnp.int32)
    u_emb, m_emb = _gather_sc(uids, mids, user_table, movie_table)
    return _mlp_tc(u_emb, m_emb, W1, b1, W2, b2, W3, b3)


# speedup vs baseline: 1.5562x; 1.5562x over previous
import functools

import jax
import jax.numpy as jnp
from jax import lax
from jax.experimental import pallas as pl
from jax.experimental.pallas import tpu as pltpu
from jax.experimental.pallas import tpu_sc as plsc

B = 16384
E = 32
H1 = 256
H2 = 64
UV = 1000001
MV = 100001
UVG = 123 * 2048
MVG = 13 * 2048


def _regroup_body(t0_ref, t1_ref, t2_ref, t3_ref, out_ref):
    out_ref[...] = jnp.concatenate(
        [jnp.swapaxes(t0_ref[...], 0, 1), jnp.swapaxes(t1_ref[...], 0, 1),
         jnp.swapaxes(t2_ref[...], 0, 1), jnp.swapaxes(t3_ref[...], 0, 1)],
        axis=1)


def _regroup_tc(tabT, vg, w):
    nblk = vg // w
    last = (tabT.shape[1] - 1) // w
    specs = [
        pl.BlockSpec((E, w),
                     lambda i, q=q: (0, jnp.minimum(q * nblk + i, last)))
        for q in range(4)
    ]
    return pl.pallas_call(
        _regroup_body,
        grid=(nblk,),
        in_specs=specs,
        out_specs=pl.BlockSpec((w, 4 * E), lambda i: (i, 0)),
        out_shape=jax.ShapeDtypeStruct((vg, 4 * E), jnp.float32),
    )(tabT, tabT, tabT, tabT)


def _gather_sc(user_ids, movie_ids, ug128, mg128):
    info = plsc.get_sparse_core_info()
    nc, ns = info.num_cores, info.num_subcores
    nw = nc * ns
    bpw = B // nw
    mesh = plsc.VectorSubcoreMesh(core_axis_name="c", subcore_axis_name="s")

    @functools.partial(
        pl.kernel,
        mesh=mesh,
        out_type=(
            jax.ShapeDtypeStruct((B, 4 * E), jnp.float32),
            jax.ShapeDtypeStruct((B, 4 * E), jnp.float32),
        ),
        scratch_types=[
            pltpu.VMEM((bpw,), jnp.int32),
            pltpu.VMEM((bpw,), jnp.int32),
            pltpu.VMEM((bpw,), jnp.int32),
            pltpu.VMEM((bpw,), jnp.int32),
            pltpu.VMEM((bpw, 4 * E), jnp.float32),
            pltpu.SemaphoreType.DMA,
        ],
        compiler_params=pltpu.CompilerParams(use_tc_tiling_on_sc=True),
    )
    def gather_kernel(uids_hbm, mids_hbm, utab_hbm, mtab_hbm, uout_hbm,
                      mout_hbm, uids_v, mids_v, uidx_v, midx_v, rows, sem):
        wid = lax.axis_index("s") * nc + lax.axis_index("c")
        base = wid * bpw
        pltpu.sync_copy(uids_hbm.at[pl.ds(base, bpw)], uids_v)
        pltpu.sync_copy(mids_hbm.at[pl.ds(base, bpw)], mids_v)

        def idx_body(j, _):
            sl = pl.ds(j * 16, 16)
            uv = uids_v[sl]
            mv = mids_v[sl]
            uq = jnp.where(uv >= UVG, 1, 0) + jnp.where(uv >= 2 * UVG, 1, 0) \
                + jnp.where(uv >= 3 * UVG, 1, 0)
            mq = jnp.where(mv >= MVG, 1, 0) + jnp.where(mv >= 2 * MVG, 1, 0) \
                + jnp.where(mv >= 3 * MVG, 1, 0)
            uidx_v[sl] = uv - uq * UVG
            midx_v[sl] = mv - mq * MVG
            return 0

        lax.fori_loop(0, bpw // 16, idx_body, 0)

        pltpu.async_copy(utab_hbm.at[uidx_v], rows, sem).wait()
        pltpu.sync_copy(rows, uout_hbm.at[pl.ds(base, bpw)])
        pltpu.async_copy(mtab_hbm.at[midx_v], rows, sem).wait()
        pltpu.sync_copy(rows, mout_hbm.at[pl.ds(base, bpw)])

    return gather_kernel(user_ids, movie_ids, ug128, mg128)


def _mlp_body(u_ref, m_ref, uid_ref, mid_ref, w1u_ref, w1m_ref, b1_ref,
              w2_ref, b2_ref, w3_ref, b3_ref, out_ref):
    uid = uid_ref[...]
    mid = mid_ref[...]
    uq = jnp.where(uid >= UVG, 1, 0) + jnp.where(uid >= 2 * UVG, 1, 0) \
        + jnp.where(uid >= 3 * UVG, 1, 0)
    mq = jnp.where(mid >= MVG, 1, 0) + jnp.where(mid >= 2 * MVG, 1, 0) \
        + jnp.where(mid >= 3 * MVG, 1, 0)
    u128 = u_ref[...]
    m128 = m_ref[...]
    xu = jnp.zeros((u128.shape[0], E), jnp.float32)
    xm = jnp.zeros((u128.shape[0], E), jnp.float32)
    for q in range(4):
        xu += jnp.where(uq == q, u128[:, q * E:(q + 1) * E], 0.0)
        xm += jnp.where(mq == q, m128[:, q * E:(q + 1) * E], 0.0)
    h = jnp.dot(xu, w1u_ref[...], preferred_element_type=jnp.float32)
    h += jnp.dot(xm, w1m_ref[...], preferred_element_type=jnp.float32)
    h = jnp.maximum(h + b1_ref[...], 0.0)
    h = jnp.dot(h, w2_ref[...], preferred_element_type=jnp.float32)
    h = jnp.maximum(h + b2_ref[...], 0.0)
    out_ref[...] = jnp.sum(h * w3_ref[...], axis=1, keepdims=True) \
        + b3_ref[...]


def _mlp_tc(u128, m128, uids2, mids2, W1, b1, W2, b2, W3, b3):
    blk = 2048
    w1u = W1[:E]
    w1m = W1[E:]
    b1r = b1.reshape(1, H1)
    b2r = b2.reshape(1, H2)
    w3r = W3.reshape(1, H2)
    b3r = b3.reshape(1, 1)
    return pl.pallas_call(
        _mlp_body,
        grid=(B // blk,),
        in_specs=[
            pl.BlockSpec((blk, 4 * E), lambda i: (i, 0)),
            pl.BlockSpec((blk, 4 * E), lambda i: (i, 0)),
            pl.BlockSpec((blk, 1), lambda i: (i, 0)),
            pl.BlockSpec((blk, 1), lambda i: (i, 0)),
            pl.BlockSpec((E, H1), lambda i: (0, 0)),
            pl.BlockSpec((E, H1), lambda i: (0, 0)),
            pl.BlockSpec((1, H1), lambda i: (0, 0)),
            pl.BlockSpec((H1, H2), lambda i: (0, 0)),
            pl.BlockSpec((1, H2), lambda i: (0, 0)),
            pl.BlockSpec((1, H2), lambda i: (0, 0)),
            pl.BlockSpec((1, 1), lambda i: (0, 0)),
        ],
        out_specs=pl.BlockSpec((blk, 1), lambda i: (i, 0)),
        out_shape=jax.ShapeDtypeStruct((B, 1), jnp.float32),
    )(u128, m128, uids2, mids2, w1u, w1m, b1r, W2, b2r, w3r, b3r)


def kernel(user_ids, movie_ids, user_table, movie_table, W1, b1, W2, b2, W3, b3):
    uids = user_ids.astype(jnp.int32)
    mids = movie_ids.astype(jnp.int32)
    ug128 = _regroup_tc(user_table.T, UVG, 2048)
    mg128 = _regroup_tc(movie_table.T, MVG, 2048)
    u128, m128 = _gather_sc(uids, mids, ug128, mg128)
    return _mlp_tc(u128, m128, uids.reshape(B, 1), mids.reshape(B, 1),
                   W1, b1, W2, b2, W3, b3)


# speedup vs baseline: 1.5782x; 1.0141x over previous
import functools

import jax
import jax.numpy as jnp
from jax import lax
from jax.experimental import pallas as pl
from jax.experimental.pallas import tpu as pltpu
from jax.experimental.pallas import tpu_sc as plsc

B = 16384
E = 32
H1 = 256
H2 = 64
UV = 1000001
MV = 100001
RGW = 4096
UVG = 62 * RGW
MVG = 7 * RGW


def _regroup_body(t0_ref, t1_ref, t2_ref, t3_ref, out_ref):
    eye = jnp.eye(E, dtype=jnp.float32)
    dn = (((0,), (0,)), ((), ()))
    out_ref[...] = jnp.concatenate(
        [lax.dot_general(r[...], eye, dn, preferred_element_type=jnp.float32)
         for r in (t0_ref, t1_ref, t2_ref, t3_ref)],
        axis=1)


def _regroup_tc(tabT, vg, w):
    nblk = vg // w
    last = (tabT.shape[1] - 1) // w
    specs = [
        pl.BlockSpec((E, w),
                     lambda i, q=q: (0, jnp.minimum(q * nblk + i, last)))
        for q in range(4)
    ]
    return pl.pallas_call(
        _regroup_body,
        grid=(nblk,),
        in_specs=specs,
        out_specs=pl.BlockSpec((w, 4 * E), lambda i: (i, 0)),
        out_shape=jax.ShapeDtypeStruct((vg, 4 * E), jnp.float32),
    )(tabT, tabT, tabT, tabT)


def _gather_sc(user_ids, movie_ids, ug128, mg128):
    info = plsc.get_sparse_core_info()
    nc, ns = info.num_cores, info.num_subcores
    nw = nc * ns
    bpw = B // nw
    mesh = plsc.VectorSubcoreMesh(core_axis_name="c", subcore_axis_name="s")

    @functools.partial(
        pl.kernel,
        mesh=mesh,
        out_type=(
            jax.ShapeDtypeStruct((B, 4 * E), jnp.float32),
            jax.ShapeDtypeStruct((B, 4 * E), jnp.float32),
        ),
        scratch_types=[
            pltpu.VMEM((bpw,), jnp.int32),
            pltpu.VMEM((bpw,), jnp.int32),
            pltpu.VMEM((bpw,), jnp.int32),
            pltpu.VMEM((bpw,), jnp.int32),
            pltpu.VMEM((bpw, 4 * E), jnp.float32),
            pltpu.SemaphoreType.DMA,
        ],
        compiler_params=pltpu.CompilerParams(use_tc_tiling_on_sc=True),
    )
    def gather_kernel(uids_hbm, mids_hbm, utab_hbm, mtab_hbm, uout_hbm,
                      mout_hbm, uids_v, mids_v, uidx_v, midx_v, rows, sem):
        wid = lax.axis_index("s") * nc + lax.axis_index("c")
        base = wid * bpw
        pltpu.sync_copy(uids_hbm.at[pl.ds(base, bpw)], uids_v)
        pltpu.sync_copy(mids_hbm.at[pl.ds(base, bpw)], mids_v)

        def idx_body(j, _):
            sl = pl.ds(j * 16, 16)
            uv = uids_v[sl]
            mv = mids_v[sl]
            uq = jnp.where(uv >= UVG, 1, 0) + jnp.where(uv >= 2 * UVG, 1, 0) \
                + jnp.where(uv >= 3 * UVG, 1, 0)
            mq = jnp.where(mv >= MVG, 1, 0) + jnp.where(mv >= 2 * MVG, 1, 0) \
                + jnp.where(mv >= 3 * MVG, 1, 0)
            uidx_v[sl] = uv - uq * UVG
            midx_v[sl] = mv - mq * MVG
            return 0

        lax.fori_loop(0, bpw // 16, idx_body, 0)

        pltpu.async_copy(utab_hbm.at[uidx_v], rows, sem).wait()
        pltpu.sync_copy(rows, uout_hbm.at[pl.ds(base, bpw)])
        pltpu.async_copy(mtab_hbm.at[midx_v], rows, sem).wait()
        pltpu.sync_copy(rows, mout_hbm.at[pl.ds(base, bpw)])

    return gather_kernel(user_ids, movie_ids, ug128, mg128)


def _mlp_body(u_ref, m_ref, uid_ref, mid_ref, w1u_ref, w1m_ref, b1_ref,
              w2_ref, b2_ref, w3_ref, b3_ref, out_ref):
    uid = uid_ref[...]
    mid = mid_ref[...]
    uq = jnp.where(uid >= UVG, 1, 0) + jnp.where(uid >= 2 * UVG, 1, 0) \
        + jnp.where(uid >= 3 * UVG, 1, 0)
    mq = jnp.where(mid >= MVG, 1, 0) + jnp.where(mid >= 2 * MVG, 1, 0) \
        + jnp.where(mid >= 3 * MVG, 1, 0)
    u128 = u_ref[...]
    m128 = m_ref[...]
    xu = jnp.zeros((u128.shape[0], E), jnp.float32)
    xm = jnp.zeros((u128.shape[0], E), jnp.float32)
    for q in range(4):
        xu += jnp.where(uq == q, u128[:, q * E:(q + 1) * E], 0.0)
        xm += jnp.where(mq == q, m128[:, q * E:(q + 1) * E], 0.0)
    h = jnp.dot(xu, w1u_ref[...], preferred_element_type=jnp.float32)
    h += jnp.dot(xm, w1m_ref[...], preferred_element_type=jnp.float32)
    h = jnp.maximum(h + b1_ref[...], 0.0)
    h = jnp.dot(h, w2_ref[...], preferred_element_type=jnp.float32)
    h = jnp.maximum(h + b2_ref[...], 0.0)
    out_ref[...] = jnp.sum(h * w3_ref[...], axis=1, keepdims=True) \
        + b3_ref[...]


def _mlp_tc(u128, m128, uids2, mids2, W1, b1, W2, b2, W3, b3):
    blk = 2048
    w1u = W1[:E]
    w1m = W1[E:]
    b1r = b1.reshape(1, H1)
    b2r = b2.reshape(1, H2)
    w3r = W3.reshape(1, H2)
    b3r = b3.reshape(1, 1)
    return pl.pallas_call(
        _mlp_body,
        grid=(B // blk,),
        in_specs=[
            pl.BlockSpec((blk, 4 * E), lambda i: (i, 0)),
            pl.BlockSpec((blk, 4 * E), lambda i: (i, 0)),
            pl.BlockSpec((blk, 1), lambda i: (i, 0)),
            pl.BlockSpec((blk, 1), lambda i: (i, 0)),
            pl.BlockSpec((E, H1), lambda i: (0, 0)),
            pl.BlockSpec((E, H1), lambda i: (0, 0)),
            pl.BlockSpec((1, H1), lambda i: (0, 0)),
            pl.BlockSpec((H1, H2), lambda i: (0, 0)),
            pl.BlockSpec((1, H2), lambda i: (0, 0)),
            pl.BlockSpec((1, H2), lambda i: (0, 0)),
            pl.BlockSpec((1, 1), lambda i: (0, 0)),
        ],
        out_specs=pl.BlockSpec((blk, 1), lambda i: (i, 0)),
        out_shape=jax.ShapeDtypeStruct((B, 1), jnp.float32),
    )(u128, m128, uids2, mids2, w1u, w1m, b1r, W2, b2r, w3r, b3r)


def kernel(user_ids, movie_ids, user_table, movie_table, W1, b1, W2, b2, W3, b3):
    uids = user_ids.astype(jnp.int32)
    mids = movie_ids.astype(jnp.int32)
    ug128 = _regroup_tc(user_table.T, UVG, RGW)
    mg128 = _regroup_tc(movie_table.T, MVG, RGW)
    u128, m128 = _gather_sc(uids, mids, ug128, mg128)
    return _mlp_tc(u128, m128, uids.reshape(B, 1), mids.reshape(B, 1),
                   W1, b1, W2, b2, W3, b3)


# speedup vs baseline: 1.7219x; 1.0910x over previous
import functools

import jax
import jax.numpy as jnp
from jax import lax
from jax.experimental import pallas as pl
from jax.experimental.pallas import tpu as pltpu
from jax.experimental.pallas import tpu_sc as plsc

B = 16384
E = 32
H1 = 256
H2 = 64

RGW = 2048
UV8 = 62 * RGW
MV8 = 7 * RGW


def _bf16_bits(x_f32):
    u = lax.bitcast_convert_type(x_f32, jnp.int32)
    rnd = lax.shift_right_logical(u, 16)
    rnd = lax.bitwise_and(rnd, 1)
    return lax.shift_right_logical(u + 0x7FFF + rnd, 16)


def _regroup_body(*refs):
    (t0, t1, t2, t3, t4, t5, t6, t7, out_ref) = refs
    eye = jnp.eye(E, dtype=jnp.float32)
    dn = (((0,), (0,)), ((), ()))
    parts = [lax.dot_general(t[...], eye, dn,
                             preferred_element_type=jnp.float32)
             for t in (t0, t1, t2, t3, t4, t5, t6, t7)]
    for p in range(4):
        lo = _bf16_bits(parts[2 * p])
        hi = _bf16_bits(parts[2 * p + 1])
        out_ref[:, p * E:(p + 1) * E] = lo | lax.shift_left(hi, 16)


def _regroup_tc(tabT, v8, w):
    nblk = v8 // w
    last = (tabT.shape[1] - 1) // w
    specs = [
        pl.BlockSpec((E, w),
                     lambda i, o=o: (0, jnp.minimum(o * nblk + i, last)))
        for o in range(8)
    ]
    return pl.pallas_call(
        _regroup_body,
        grid=(nblk,),
        in_specs=specs,
        out_specs=pl.BlockSpec((w, 4 * E), lambda i: (i, 0)),
        out_shape=jax.ShapeDtypeStruct((v8, 4 * E), jnp.int32),
    )(*([tabT] * 8))


def _octant(v, v8):
    q = jnp.zeros_like(v)
    for o in range(1, 8):
        q += jnp.where(v >= o * v8, 1, 0)
    return q


def _gather_sc(user_ids, movie_ids, ug, mg):
    info = plsc.get_sparse_core_info()
    nc, ns = info.num_cores, info.num_subcores
    nw = nc * ns
    bpw = B // nw
    mesh = plsc.VectorSubcoreMesh(core_axis_name="c", subcore_axis_name="s")

    @functools.partial(
        pl.kernel,
        mesh=mesh,
        out_type=(
            jax.ShapeDtypeStruct((B, 4 * E), jnp.int32),
            jax.ShapeDtypeStruct((B, 4 * E), jnp.int32),
        ),
        scratch_types=[
            pltpu.VMEM((bpw,), jnp.int32),
            pltpu.VMEM((bpw,), jnp.int32),
            pltpu.VMEM((bpw,), jnp.int32),
            pltpu.VMEM((bpw,), jnp.int32),
            pltpu.VMEM((bpw, 4 * E), jnp.int32),
            pltpu.SemaphoreType.DMA,
        ],
        compiler_params=pltpu.CompilerParams(use_tc_tiling_on_sc=True),
    )
    def gather_kernel(uids_hbm, mids_hbm, utab_hbm, mtab_hbm, uout_hbm,
                      mout_hbm, uids_v, mids_v, uidx_v, midx_v, rows, sem):
        wid = lax.axis_index("s") * nc + lax.axis_index("c")
        base = wid * bpw
        pltpu.sync_copy(uids_hbm.at[pl.ds(base, bpw)], uids_v)
        pltpu.sync_copy(mids_hbm.at[pl.ds(base, bpw)], mids_v)

        def idx_body(j, _):
            sl = pl.ds(j * 16, 16)
            uv = uids_v[sl]
            mv = mids_v[sl]
            uidx_v[sl] = uv - _octant(uv, UV8) * UV8
            midx_v[sl] = mv - _octant(mv, MV8) * MV8
            return 0

        lax.fori_loop(0, bpw // 16, idx_body, 0)
        pltpu.async_copy(utab_hbm.at[uidx_v], rows, sem).wait()
        pltpu.sync_copy(rows, uout_hbm.at[pl.ds(base, bpw)])
        pltpu.async_copy(mtab_hbm.at[midx_v], rows, sem).wait()
        pltpu.sync_copy(rows, mout_hbm.at[pl.ds(base, bpw)])

    return gather_kernel(user_ids, movie_ids, ug, mg)


def _select_bf16(x128, v, v8):
    o = _octant(v, v8)
    p = lax.shift_right_logical(o, 1)
    h = lax.bitwise_and(o, 1)
    word = jnp.zeros((x128.shape[0], E), jnp.int32)
    for pp in range(4):
        word += jnp.where(p == pp, x128[:, pp * E:(pp + 1) * E], 0)
    bits = jnp.where(h == 1, lax.shift_right_logical(word, 16), word)
    bits = lax.shift_left(bits, 16)
    return lax.bitcast_convert_type(bits, jnp.float32).astype(jnp.bfloat16)


def _mlp_body(u_ref, m_ref, uid_ref, mid_ref, w1u_ref, w1m_ref, b1_ref,
              w2_ref, b2_ref, w3_ref, b3_ref, out_ref):
    xu = _select_bf16(u_ref[...], uid_ref[...], UV8)
    xm = _select_bf16(m_ref[...], mid_ref[...], MV8)
    h = jnp.dot(xu, w1u_ref[...], preferred_element_type=jnp.float32)
    h += jnp.dot(xm, w1m_ref[...], preferred_element_type=jnp.float32)
    h = jnp.maximum(h + b1_ref[...], 0.0)
    h = jnp.dot(h, w2_ref[...], preferred_element_type=jnp.float32)
    h = jnp.maximum(h + b2_ref[...], 0.0)
    out_ref[...] = jnp.sum(h * w3_ref[...], axis=1, keepdims=True) \
        + b3_ref[...]


def _mlp_tc(u128, m128, uids2, mids2, W1, b1, W2, b2, W3, b3):
    blk = 2048
    w1u = W1[:E].astype(jnp.bfloat16)
    w1m = W1[E:].astype(jnp.bfloat16)
    b1r = b1.reshape(1, H1)
    b2r = b2.reshape(1, H2)
    w3r = W3.reshape(1, H2)
    b3r = b3.reshape(1, 1)
    return pl.pallas_call(
        _mlp_body,
        grid=(B // blk,),
        in_specs=[
            pl.BlockSpec((blk, 4 * E), lambda i: (i, 0)),
            pl.BlockSpec((blk, 4 * E), lambda i: (i, 0)),
            pl.BlockSpec((blk, 1), lambda i: (i, 0)),
            pl.BlockSpec((blk, 1), lambda i: (i, 0)),
            pl.BlockSpec((E, H1), lambda i: (0, 0)),
            pl.BlockSpec((E, H1), lambda i: (0, 0)),
            pl.BlockSpec((1, H1), lambda i: (0, 0)),
            pl.BlockSpec((H1, H2), lambda i: (0, 0)),
            pl.BlockSpec((1, H2), lambda i: (0, 0)),
            pl.BlockSpec((1, H2), lambda i: (0, 0)),
            pl.BlockSpec((1, 1), lambda i: (0, 0)),
        ],
        out_specs=pl.BlockSpec((blk, 1), lambda i: (i, 0)),
        out_shape=jax.ShapeDtypeStruct((B, 1), jnp.float32),
    )(u128, m128, uids2, mids2, w1u, w1m, b1r, W2, b2r, w3r, b3r)


def kernel(user_ids, movie_ids, user_table, movie_table, W1, b1, W2, b2, W3, b3):
    uids = user_ids.astype(jnp.int32)
    mids = movie_ids.astype(jnp.int32)
    ug = _regroup_tc(user_table.T, UV8, RGW)
    mg = _regroup_tc(movie_table.T, MV8, RGW)
    u128, m128 = _gather_sc(uids, mids, ug, mg)
    return _mlp_tc(u128, m128, uids.reshape(B, 1), mids.reshape(B, 1),
                   W1, b1, W2, b2, W3, b3)


# speedup vs baseline: 1.8042x; 1.0478x over previous
import functools

import jax
import jax.numpy as jnp
from jax import lax
from jax.experimental import pallas as pl
from jax.experimental.pallas import tpu as pltpu
from jax.experimental.pallas import tpu_sc as plsc

B = 16384
E = 32
H1 = 256
H2 = 64

RGW = 2048
UV8 = 62 * RGW
MV8 = 7 * RGW


def _regroup_body(*refs):
    (t0, t1, t2, t3, t4, t5, t6, t7, out_ref) = refs
    eye = jnp.eye(E, dtype=jnp.float32)
    dn = (((0,), (0,)), ((), ()))
    parts = [lax.dot_general(t[...], eye, dn,
                             preferred_element_type=jnp.float32)
             for t in (t0, t1, t2, t3, t4, t5, t6, t7)]
    for p in range(4):
        lo = lax.bitcast_convert_type(parts[2 * p], jnp.int32)
        hi = lax.bitcast_convert_type(parts[2 * p + 1], jnp.int32)
        out_ref[:, p * E:(p + 1) * E] = (
            lax.shift_right_logical(lo, 16)
            | lax.bitwise_and(hi, jnp.int32(-65536)))


def _regroup_tc(tabT, v8, w):
    nblk = v8 // w
    last = (tabT.shape[1] - 1) // w
    specs = [
        pl.BlockSpec((E, w),
                     lambda i, o=o: (0, jnp.minimum(o * nblk + i, last)))
        for o in range(8)
    ]
    return pl.pallas_call(
        _regroup_body,
        grid=(nblk,),
        in_specs=specs,
        out_specs=pl.BlockSpec((w, 4 * E), lambda i: (i, 0)),
        out_shape=jax.ShapeDtypeStruct((v8, 4 * E), jnp.int32),
    )(*([tabT] * 8))


def _octant(v, v8):
    q = jnp.zeros_like(v)
    for o in range(1, 8):
        q += jnp.where(v >= o * v8, 1, 0)
    return q


def _gather_sc(user_ids, movie_ids, ug, mg):
    info = plsc.get_sparse_core_info()
    nc, ns = info.num_cores, info.num_subcores
    nw = nc * ns
    bpw = B // nw
    mesh = plsc.VectorSubcoreMesh(core_axis_name="c", subcore_axis_name="s")

    @functools.partial(
        pl.kernel,
        mesh=mesh,
        out_type=(
            jax.ShapeDtypeStruct((B, 4 * E), jnp.int32),
            jax.ShapeDtypeStruct((B, 4 * E), jnp.int32),
        ),
        scratch_types=[
            pltpu.VMEM((bpw,), jnp.int32),
            pltpu.VMEM((bpw,), jnp.int32),
            pltpu.VMEM((bpw,), jnp.int32),
            pltpu.VMEM((bpw,), jnp.int32),
            pltpu.VMEM((bpw, 4 * E), jnp.int32),
            pltpu.SemaphoreType.DMA,
        ],
        compiler_params=pltpu.CompilerParams(use_tc_tiling_on_sc=True),
    )
    def gather_kernel(uids_hbm, mids_hbm, utab_hbm, mtab_hbm, uout_hbm,
                      mout_hbm, uids_v, mids_v, uidx_v, midx_v, rows, sem):
        wid = lax.axis_index("s") * nc + lax.axis_index("c")
        base = wid * bpw
        pltpu.sync_copy(uids_hbm.at[pl.ds(base, bpw)], uids_v)
        pltpu.sync_copy(mids_hbm.at[pl.ds(base, bpw)], mids_v)

        def idx_body(j, _):
            sl = pl.ds(j * 16, 16)
            uv = uids_v[sl]
            mv = mids_v[sl]
            uidx_v[sl] = uv - _octant(uv, UV8) * UV8
            midx_v[sl] = mv - _octant(mv, MV8) * MV8
            return 0

        lax.fori_loop(0, bpw // 16, idx_body, 0)
        pltpu.async_copy(utab_hbm.at[uidx_v], rows, sem).wait()
        pltpu.sync_copy(rows, uout_hbm.at[pl.ds(base, bpw)])
        pltpu.async_copy(mtab_hbm.at[midx_v], rows, sem).wait()
        pltpu.sync_copy(rows, mout_hbm.at[pl.ds(base, bpw)])

    return gather_kernel(user_ids, movie_ids, ug, mg)


def _select_bf16(x128, v, v8):
    o = _octant(v, v8)
    p = lax.shift_right_logical(o, 1)
    h = lax.bitwise_and(o, 1)
    word = jnp.zeros((x128.shape[0], E), jnp.int32)
    for pp in range(4):
        word += jnp.where(p == pp, x128[:, pp * E:(pp + 1) * E], 0)
    bits = jnp.where(h == 1, lax.shift_right_logical(word, 16), word)
    bits = lax.shift_left(bits, 16)
    return lax.bitcast_convert_type(bits, jnp.float32).astype(jnp.bfloat16)


def _mlp_body(u_ref, m_ref, uid_ref, mid_ref, w1u_ref, w1m_ref, b1_ref,
              w2_ref, b2_ref, w3_ref, b3_ref, out_ref):
    xu = _select_bf16(u_ref[...], uid_ref[...], UV8)
    xm = _select_bf16(m_ref[...], mid_ref[...], MV8)
    h = jnp.dot(xu, w1u_ref[...], preferred_element_type=jnp.float32)
    h += jnp.dot(xm, w1m_ref[...], preferred_element_type=jnp.float32)
    h = jnp.maximum(h + b1_ref[...], 0.0)
    h = jnp.dot(h, w2_ref[...], preferred_element_type=jnp.float32)
    h = jnp.maximum(h + b2_ref[...], 0.0)
    out_ref[...] = jnp.sum(h * w3_ref[...], axis=1, keepdims=True) \
        + b3_ref[...]


def _mlp_tc(u128, m128, uids2, mids2, W1, b1, W2, b2, W3, b3):
    blk = 2048
    w1u = W1[:E].astype(jnp.bfloat16)
    w1m = W1[E:].astype(jnp.bfloat16)
    b1r = b1.reshape(1, H1)
    b2r = b2.reshape(1, H2)
    w3r = W3.reshape(1, H2)
    b3r = b3.reshape(1, 1)
    return pl.pallas_call(
        _mlp_body,
        grid=(B // blk,),
        in_specs=[
            pl.BlockSpec((blk, 4 * E), lambda i: (i, 0)),
            pl.BlockSpec((blk, 4 * E), lambda i: (i, 0)),
            pl.BlockSpec((blk, 1), lambda i: (i, 0)),
            pl.BlockSpec((blk, 1), lambda i: (i, 0)),
            pl.BlockSpec((E, H1), lambda i: (0, 0)),
            pl.BlockSpec((E, H1), lambda i: (0, 0)),
            pl.BlockSpec((1, H1), lambda i: (0, 0)),
            pl.BlockSpec((H1, H2), lambda i: (0, 0)),
            pl.BlockSpec((1, H2), lambda i: (0, 0)),
            pl.BlockSpec((1, H2), lambda i: (0, 0)),
            pl.BlockSpec((1, 1), lambda i: (0, 0)),
        ],
        out_specs=pl.BlockSpec((blk, 1), lambda i: (i, 0)),
        out_shape=jax.ShapeDtypeStruct((B, 1), jnp.float32),
    )(u128, m128, uids2, mids2, w1u, w1m, b1r, W2, b2r, w3r, b3r)


def kernel(user_ids, movie_ids, user_table, movie_table, W1, b1, W2, b2, W3, b3):
    uids = user_ids.astype(jnp.int32)
    mids = movie_ids.astype(jnp.int32)
    ug = _regroup_tc(user_table.T, UV8, 2 * RGW)
    mg = _regroup_tc(movie_table.T, MV8, RGW)
    u128, m128 = _gather_sc(uids, mids, ug, mg)
    return _mlp_tc(u128, m128, uids.reshape(B, 1), mids.reshape(B, 1),
                   W1, b1, W2, b2, W3, b3)


# speedup vs baseline: 1.8303x; 1.0144x over previous
import functools

import jax
import jax.numpy as jnp
from jax import lax
from jax.experimental import pallas as pl
from jax.experimental.pallas import tpu as pltpu
from jax.experimental.pallas import tpu_sc as plsc

B = 16384
E = 32
H1 = 256
H2 = 64

RGW = 2048
UV8 = 62 * RGW
MV8 = 7 * RGW


def _regroup_body(*refs):
    (t0, t1, t2, t3, t4, t5, t6, t7, out_ref) = refs
    eye = jnp.eye(E, dtype=jnp.float32)
    dn = (((0,), (0,)), ((), ()))
    parts = [lax.dot_general(t[...], eye, dn,
                             preferred_element_type=jnp.float32)
             for t in (t0, t1, t2, t3, t4, t5, t6, t7)]
    for p in range(4):
        lo = lax.bitcast_convert_type(parts[2 * p], jnp.int32)
        hi = lax.bitcast_convert_type(parts[2 * p + 1], jnp.int32)
        out_ref[:, p * E:(p + 1) * E] = (
            lax.shift_right_logical(lo, 16)
            | lax.bitwise_and(hi, jnp.int32(-65536)))


def _regroup_tc(tabT, v8, w):
    nblk = v8 // w
    last = (tabT.shape[1] - 1) // w
    specs = [
        pl.BlockSpec((E, w),
                     lambda i, o=o: (0, jnp.minimum(o * nblk + i, last)))
        for o in range(8)
    ]
    return pl.pallas_call(
        _regroup_body,
        grid=(nblk,),
        in_specs=specs,
        out_specs=pl.BlockSpec((w, 4 * E), lambda i: (i, 0)),
        out_shape=jax.ShapeDtypeStruct((v8, 4 * E), jnp.int32),
    )(*([tabT] * 8))


def _octant(v, v8):
    q = jnp.zeros_like(v)
    for o in range(1, 8):
        q += jnp.where(v >= o * v8, 1, 0)
    return q


def _gather_sc(ids, tab, v8):
    info = plsc.get_sparse_core_info()
    nc, ns = info.num_cores, info.num_subcores
    nw = nc * ns
    bpw = B // nw
    mesh = plsc.VectorSubcoreMesh(core_axis_name="c", subcore_axis_name="s")

    @functools.partial(
        pl.kernel,
        mesh=mesh,
        out_type=jax.ShapeDtypeStruct((B, 4 * E), jnp.int32),
        scratch_types=[
            pltpu.VMEM((bpw,), jnp.int32),
            pltpu.VMEM((bpw,), jnp.int32),
            pltpu.VMEM((bpw, 4 * E), jnp.int32),
            pltpu.SemaphoreType.DMA,
        ],
        compiler_params=pltpu.CompilerParams(use_tc_tiling_on_sc=True),
    )
    def gather_kernel(ids_hbm, tab_hbm, out_hbm, ids_v, idx_v, rows, sem):
        wid = lax.axis_index("s") * nc + lax.axis_index("c")
        base = wid * bpw
        pltpu.sync_copy(ids_hbm.at[pl.ds(base, bpw)], ids_v)

        def idx_body(j, _):
            sl = pl.ds(j * 16, 16)
            v = ids_v[sl]
            idx_v[sl] = v - _octant(v, v8) * v8
            return 0

        lax.fori_loop(0, bpw // 16, idx_body, 0)
        pltpu.async_copy(tab_hbm.at[idx_v], rows, sem).wait()
        pltpu.sync_copy(rows, out_hbm.at[pl.ds(base, bpw)])

    return gather_kernel(ids, tab)


def _select_bf16(x128, v, v8):
    o = _octant(v, v8)
    p = lax.shift_right_logical(o, 1)
    h = lax.bitwise_and(o, 1)
    word = jnp.zeros((x128.shape[0], E), jnp.int32)
    for pp in range(4):
        word += jnp.where(p == pp, x128[:, pp * E:(pp + 1) * E], 0)
    bits = jnp.where(h == 1, lax.shift_right_logical(word, 16), word)
    bits = lax.shift_left(bits, 16)
    return lax.bitcast_convert_type(bits, jnp.float32).astype(jnp.bfloat16)


def _mlp_body(u_ref, m_ref, uid_ref, mid_ref, w1u_ref, w1m_ref, b1_ref,
              w2_ref, b2_ref, w3_ref, b3_ref, out_ref):
    xu = _select_bf16(u_ref[...], uid_ref[...], UV8)
    xm = _select_bf16(m_ref[...], mid_ref[...], MV8)
    h = jnp.dot(xu, w1u_ref[...], preferred_element_type=jnp.float32)
    h += jnp.dot(xm, w1m_ref[...], preferred_element_type=jnp.float32)
    h = jnp.maximum(h + b1_ref[...], 0.0)
    h = jnp.dot(h, w2_ref[...], preferred_element_type=jnp.float32)
    h = jnp.maximum(h + b2_ref[...], 0.0)
    out_ref[...] = jnp.sum(h * w3_ref[...], axis=1, keepdims=True) \
        + b3_ref[...]


def _mlp_tc(u128, m128, uids2, mids2, W1, b1, W2, b2, W3, b3):
    blk = 2048
    w1u = W1[:E].astype(jnp.bfloat16)
    w1m = W1[E:].astype(jnp.bfloat16)
    b1r = b1.reshape(1, H1)
    b2r = b2.reshape(1, H2)
    w3r = W3.reshape(1, H2)
    b3r = b3.reshape(1, 1)
    return pl.pallas_call(
        _mlp_body,
        grid=(B // blk,),
        in_specs=[
            pl.BlockSpec((blk, 4 * E), lambda i: (i, 0)),
            pl.BlockSpec((blk, 4 * E), lambda i: (i, 0)),
            pl.BlockSpec((blk, 1), lambda i: (i, 0)),
            pl.BlockSpec((blk, 1), lambda i: (i, 0)),
            pl.BlockSpec((E, H1), lambda i: (0, 0)),
            pl.BlockSpec((E, H1), lambda i: (0, 0)),
            pl.BlockSpec((1, H1), lambda i: (0, 0)),
            pl.BlockSpec((H1, H2), lambda i: (0, 0)),
            pl.BlockSpec((1, H2), lambda i: (0, 0)),
            pl.BlockSpec((1, H2), lambda i: (0, 0)),
            pl.BlockSpec((1, 1), lambda i: (0, 0)),
        ],
        out_specs=pl.BlockSpec((blk, 1), lambda i: (i, 0)),
        out_shape=jax.ShapeDtypeStruct((B, 1), jnp.float32),
    )(u128, m128, uids2, mids2, w1u, w1m, b1r, W2, b2r, w3r, b3r)


def kernel(user_ids, movie_ids, user_table, movie_table, W1, b1, W2, b2, W3, b3):
    uids = user_ids.astype(jnp.int32)
    mids = movie_ids.astype(jnp.int32)
    mg = _regroup_tc(movie_table.T, MV8, RGW)
    m128 = _gather_sc(mids, mg, MV8)
    ug = _regroup_tc(user_table.T, UV8, 2 * RGW)
    u128 = _gather_sc(uids, ug, UV8)
    return _mlp_tc(u128, m128, uids.reshape(B, 1), mids.reshape(B, 1),
                   W1, b1, W2, b2, W3, b3)
